# baseline (speedup 1.0000x reference)
import jax
import jax.numpy as jnp
from jax import lax
from jax.experimental import pallas as pl
from jax.experimental.pallas import tpu as pltpu
from jax.experimental.pallas import tpu_sc as plsc

H = 16
S = 2048
NPOS = 2 * S - 1
PAD = NPOS + 1
NSHIFT = 8
SLAB = 8 * S
NSLAB = S // 2 // 8


def _body(tbl_hbm, out_hbm, tbl_v, rev_v, slab_v, sem):
    h = lax.axis_index("s")
    half = lax.axis_index("c")
    pltpu.sync_copy(tbl_hbm.at[h], tbl_v)
    lanes = lax.iota(jnp.int32, 16)

    @plsc.parallel_loop(0, NSHIFT * (PAD // 16), unroll=8)
    def build(t):
        s = lax.shift_right_logical(t, 8)
        m = lax.bitwise_and(t, (PAD // 16) - 1) * 16
        k = jnp.maximum((NPOS - 1) - s - (m + lanes), 0)
        idx = [lax.shift_right_logical(k, 10),
               lax.bitwise_and(lax.shift_right_logical(k, 7), 7),
               lax.bitwise_and(k, 127)]
        rev_v[pl.ds(t * 16, 16)] = plsc.load_gather(tbl_v, idx)

    si0 = half * NSLAB

    def stage(i, b):
        a00 = (S - 1) - 8 * (si0 + i)

        @plsc.parallel_loop(0, SLAB // 16, unroll=16)
        def piece(w):
            u = lax.shift_right_logical(w, 7)
            v = lax.bitwise_and(w, 127) * 16
            a = a00 - u + v
            s = lax.bitwise_and(a, NSHIFT - 1)
            off = pl.multiple_of(s * PAD + a - s, NSHIFT)
            slab_v[b, u, pl.ds(v, 16)] = rev_v[pl.ds(off, 16)]

    def fire(i, b):
        pltpu.make_async_copy(
            slab_v.at[b], out_hbm.at[h, si0 + i], sem).start()

    def wait_one():
        pltpu.make_async_copy(
            slab_v.at[0], out_hbm.at[h, si0], sem).wait()

    for p in range(4):
        stage(p, p)
        fire(p, p)

    def steady(i, c):
        b = lax.bitwise_and(i, 3)
        wait_one()
        stage(i, b)
        fire(i, b)
        return c

    lax.fori_loop(4, NSLAB, steady, 0)
    for p in range(4):
        wait_one()


def kernel(relative_bias, seq_len):
    del seq_len
    tbl4 = jnp.pad(relative_bias, ((0, 0), (0, 1))).reshape(H, 4, 8, 128)
    mesh = plsc.VectorSubcoreMesh(core_axis_name="c", subcore_axis_name="s")
    f = pl.kernel(
        _body,
        out_type=jax.ShapeDtypeStruct((H, S // 8, 8, S), jnp.float32),
        mesh=mesh,
        scratch_types=[
            pltpu.VMEM((4, 8, 128), jnp.float32),
            pltpu.VMEM((NSHIFT * PAD,), jnp.float32),
            pltpu.VMEM((4, 8, S), jnp.float32),
            pltpu.SemaphoreType.DMA,
        ],
        compiler_params=pltpu.CompilerParams(
            needs_layout_passes=False, use_tc_tiling_on_sc=True,
            skip_device_barrier=True),
    )
    return f(tbl4).reshape(H, S, S)


# speedup vs baseline: 1.6328x; 1.6328x over previous
import jax
import jax.numpy as jnp
from jax import lax
from jax.experimental import pallas as pl
from jax.experimental.pallas import tpu as pltpu
from jax.experimental.pallas import tpu_sc as plsc

H = 16
S = 2048
NPOS = 2 * S - 1
PAD = NPOS + 1
NSHIFT = 8
SLAB = 8 * S
NSLAB = S // 2 // 8


def _body(tbl_hbm, out_hbm, tbl_v, rev_v, slab_v, sem):
    h = lax.axis_index("s")
    half = lax.axis_index("c")
    pltpu.sync_copy(tbl_hbm.at[h], tbl_v)
    lanes = lax.iota(jnp.int32, 16)

    @plsc.parallel_loop(0, NSHIFT * (PAD // 16), unroll=8)
    def build(t):
        s = lax.shift_right_logical(t, 8)
        m = lax.bitwise_and(t, (PAD // 16) - 1) * 16
        k = jnp.maximum((NPOS - 1) - s - (m + lanes), 0)
        idx = [lax.shift_right_logical(k, 10),
               lax.bitwise_and(lax.shift_right_logical(k, 7), 7),
               lax.bitwise_and(k, 127)]
        rev_v[pl.ds(t * 16, 16)] = plsc.load_gather(tbl_v, idx)

    si0 = half * NSLAB

    def stage(i, b):
        a00 = (S - 1) - 8 * (si0 + i)

        @plsc.parallel_loop(0, SLAB // 16, unroll=8)
        def piece(w):
            u = lax.shift_right_logical(w, 7)
            v = lax.bitwise_and(w, 127) * 16
            a = a00 - u + v
            s = lax.bitwise_and(a, NSHIFT - 1)
            off = pl.multiple_of(s * PAD + a - s, NSHIFT)
            slab_v[b, u, pl.ds(v, 16)] = rev_v[pl.ds(off, 16)]

    def fire(i, b):
        pltpu.make_async_copy(
            slab_v.at[b], out_hbm.at[h, si0 + i], sem).start()

    def wait_one():
        pltpu.make_async_copy(
            slab_v.at[0], out_hbm.at[h, si0], sem).wait()

    stage(0, 0)
    fire(0, 0)
    stage(1, 1)
    fire(1, 1)

    def steady(i, c):
        b = lax.bitwise_and(i, 1)
        wait_one()
        stage(i, b)
        fire(i, b)
        return c

    lax.fori_loop(2, NSLAB, steady, 0)
    wait_one()
    wait_one()


def kernel(relative_bias, seq_len):
    del seq_len
    tbl4 = jnp.pad(relative_bias, ((0, 0), (0, 1))).reshape(H, 4, 8, 128)
    mesh = plsc.VectorSubcoreMesh(core_axis_name="c", subcore_axis_name="s")
    f = pl.kernel(
        _body,
        out_type=jax.ShapeDtypeStruct((H, S // 8, 8, S), jnp.float32),
        mesh=mesh,
        scratch_types=[
            pltpu.VMEM((4, 8, 128), jnp.float32),
            pltpu.VMEM((NSHIFT * PAD,), jnp.float32),
            pltpu.VMEM((2, 8, S), jnp.float32),
            pltpu.SemaphoreType.DMA,
        ],
        compiler_params=pltpu.CompilerParams(
            needs_layout_passes=False, use_tc_tiling_on_sc=True,
            skip_device_barrier=True),
    )
    return f(tbl4).reshape(H, S, S)
